# Initial kernel scaffold; baseline (speedup 1.0000x reference)
import jax
import jax.numpy as jnp
from jax.experimental import pallas as pl


def kernel(node_features, positions, senders, receivers, W_pre_s, W_pre_v, W_post_s, W_post_v, W_sc):
    raise NotImplementedError("write your pallas kernel here")


# speedup vs baseline: 9.8479x; 9.8479x over previous
import jax
import jax.numpy as jnp
from jax import lax
from jax.experimental import pallas as pl
from jax.experimental.pallas import tpu as pltpu
from jax.experimental.pallas import tpu_sc as plsc

N_NODES = 10000
N_EDGES = 320000
D = 128
DH = 64

N_TILES = 16
EPT = 20480
E_PAD = EPT * N_TILES
BLK = 128
NBLK = EPT // BLK
N_PAD = N_NODES + 8
RPT = 632
RPT_LAST = N_NODES - 15 * RPT

_SQRT3 = 3.0 ** 0.5


def _rsqrt(x):
    i = lax.bitcast_convert_type(x, jnp.int32)
    i = jnp.int32(0x5F3759DF) - (i >> 1)
    y = lax.bitcast_convert_type(i, jnp.float32)
    for _ in range(3):
        y = y * (1.5 - 0.5 * x * y * y)
    return y


def _sc_body(nf0_hbm, nf1_hbm, px_hbm, py_hbm, pz_hbm, snd_hbm, rcv_hbm,
             o00, o10, o01, o11, o20, o30, o21, o31,
             px_v, py_v, pz_v, sidx_v, ridx_v, wbuf, rows_v, wrow_v, zbuf,
             acc, sem):
    cid = lax.axis_index("c")
    sid = lax.axis_index("s")
    sweeps = ((0, o00, o10), (1, o01, o11), (0, o20, o30), (1, o21, o31))

    pltpu.sync_copy(px_hbm, px_v)
    pltpu.sync_copy(py_hbm, py_v)
    pltpu.sync_copy(pz_hbm, pz_v)

    def _zero_row(i, _):
        for j in range(DH // 16):
            zbuf[i, pl.ds(j * 16, 16)] = jnp.zeros((16,), jnp.float32)
        return 0
    lax.fori_loop(0, 128, _zero_row, 0)

    zeros16i = jnp.zeros((16,), jnp.int32)
    ones16 = jnp.ones((16,), jnp.float32)
    r0 = pl.multiple_of(sid * RPT, 8)

    for swp, (half, out_a, out_b) in enumerate(sweeps):
        nf_hbm = nf0_hbm if half == 0 else nf1_hbm

        for j in range(4):
            pltpu.sync_copy(zbuf, acc.at[pl.ds(r0 + j * 128, 128)])

        @pl.when(sid < N_TILES - 1)
        def _():
            pltpu.sync_copy(zbuf.at[pl.ds(0, RPT - 512)],
                            acc.at[pl.ds(r0 + 512, RPT - 512)])

        @pl.when(sid == N_TILES - 1)
        def _():
            pltpu.sync_copy(zbuf.at[pl.ds(0, N_PAD - 15 * RPT - 512)],
                            acc.at[pl.ds(15 * RPT + 512, N_PAD - 15 * RPT - 512)])

        plsc.subcore_barrier()

        def _block(b, _):
            e0 = pl.multiple_of(sid * EPT + b * BLK, 128)
            pltpu.sync_copy(snd_hbm.at[pl.ds(e0, BLK)], sidx_v)
            pltpu.sync_copy(rcv_hbm.at[pl.ds(e0, BLK)], ridx_v.at[0])
            gather = pltpu.async_copy(nf_hbm.at[sidx_v], rows_v, sem)

            for i in range(BLK // 16):
                s16 = sidx_v[pl.ds(i * 16, 16)]
                r16 = ridx_v[0, pl.ds(i * 16, 16)]
                sx = plsc.load_gather(px_v, [s16])
                sy = plsc.load_gather(py_v, [s16])
                sz = plsc.load_gather(pz_v, [s16])
                rx = plsc.load_gather(px_v, [r16])
                ry = plsc.load_gather(py_v, [r16])
                rz = plsc.load_gather(pz_v, [r16])
                vx, vy, vz = rx - sx, ry - sy, rz - sz
                rinv = _rsqrt(vx * vx + vy * vy + vz * vz + 1e-12) * _SQRT3
                if swp < 2:
                    w16 = jnp.where(cid == 0, ones16, vx * rinv)
                else:
                    w16 = jnp.where(cid == 0, vy * rinv, vz * rinv)
                wbuf[pl.ds(i * 16, 16)] = w16

            gather.wait()

            def _mul(e, _):
                wsp = plsc.load_gather(wbuf, [zeros16i + e])
                for j in range(DH // 16):
                    wrow_v[e, pl.ds(j * 16, 16)] = wsp * rows_v[e, pl.ds(j * 16, 16)]
                return 0
            lax.fori_loop(0, BLK, _mul, 0)

            pltpu.sync_copy(wrow_v, acc.at[ridx_v.at[0]], add=True)
            return 0

        lax.fori_loop(0, NBLK, _block, 0)

        plsc.subcore_barrier()

        for this_cid, out_x in ((0, out_a), (1, out_b)):
            @pl.when((cid == this_cid) & (sid < N_TILES - 1))
            def _(out_x=out_x):
                pltpu.sync_copy(acc.at[pl.ds(r0, RPT)],
                                out_x.at[pl.ds(r0, RPT)])

            @pl.when((cid == this_cid) & (sid == N_TILES - 1))
            def _(out_x=out_x):
                pltpu.sync_copy(acc.at[pl.ds(15 * RPT, RPT_LAST)],
                                out_x.at[pl.ds(15 * RPT, RPT_LAST)])


def _sc_aggregate(node_features, positions, senders, receivers):
    pad = E_PAD - N_EDGES
    snd_p = jnp.concatenate([senders, jnp.zeros((pad,), jnp.int32)])
    rcv_p = jnp.concatenate([receivers, jnp.full((pad,), N_NODES, jnp.int32)])

    nf0 = node_features[:, :DH]
    nf1 = node_features[:, DH:]
    px, py, pz = positions[:, 0], positions[:, 1], positions[:, 2]
    mesh = plsc.VectorSubcoreMesh(core_axis_name="c", subcore_axis_name="s")
    f32 = jnp.float32
    agg_shape = jax.ShapeDtypeStruct((N_NODES, DH), f32)
    fn = pl.kernel(
        _sc_body,
        mesh=mesh,
        compiler_params=pltpu.CompilerParams(
            needs_layout_passes=False, use_tc_tiling_on_sc=False),
        out_type=tuple(agg_shape for _ in range(8)),
        scratch_types=[
            pltpu.VMEM((N_NODES,), f32),
            pltpu.VMEM((N_NODES,), f32),
            pltpu.VMEM((N_NODES,), f32),
            pltpu.VMEM((BLK,), jnp.int32),
            pltpu.VMEM((1, BLK), jnp.int32),
            pltpu.VMEM((BLK,), f32),
            pltpu.VMEM((BLK, DH), f32),
            pltpu.VMEM((BLK, DH), f32),
            pltpu.VMEM((128, DH), f32),
            pltpu.VMEM_SHARED((N_PAD, DH), f32),
            pltpu.SemaphoreType.DMA,
        ],
    )
    o00, o10, o01, o11, o20, o30, o21, o31 = fn(
        nf0, nf1, px, py, pz, snd_p, rcv_p)
    return ((o00, o01), (o10, o11), (o20, o21), (o30, o31))


def _tc_body(a0l_ref, a0h_ref, a1l_ref, a1h_ref, a2l_ref, a2h_ref,
             a3l_ref, a3h_ref, nf_ref,
             wps_ref, wpv_ref, wos_ref, wov_ref, wsc_ref, out_ref):
    inv = 1.0 / (D ** 0.5)
    den = 1.0 / 32.0
    f32 = jnp.float32

    def matmul_split(lo, hi, w_ref):
        return (jnp.dot(lo, w_ref[0:DH, :], preferred_element_type=f32)
                + jnp.dot(hi, w_ref[DH:D, :], preferred_element_type=f32))

    s1 = matmul_split(a0l_ref[...] * den, a0h_ref[...] * den, wps_ref) * inv
    s1 = s1 * jax.nn.sigmoid(s1)
    s2 = jnp.dot(s1, wos_ref[...], preferred_element_type=f32) * inv
    sc = jnp.dot(nf_ref[...], wsc_ref[...], preferred_element_type=f32) * inv
    out_ref[:, 0:D] = sc + s2

    rows = lax.broadcasted_iota(jnp.int32, (D, 3 * D), 0)
    cols = lax.broadcasted_iota(jnp.int32, (D, 3 * D), 1)
    outv = jnp.zeros((out_ref.shape[0], 3 * D), f32)
    for i, (lo_ref, hi_ref) in enumerate(
            ((a1l_ref, a1h_ref), (a2l_ref, a2h_ref), (a3l_ref, a3h_ref))):
        v1 = matmul_split(lo_ref[...] * den, hi_ref[...] * den, wpv_ref) * inv
        v2 = jnp.dot(v1, wov_ref[...], preferred_element_type=f32) * inv
        perm = (cols == 3 * rows + i).astype(f32)
        outv = outv + jnp.dot(v2, perm, preferred_element_type=f32)
    out_ref[:, D:4 * D] = outv


def _tc_update(aggs, node_features, W_pre_s, W_pre_v, W_post_s, W_post_v, W_sc):
    bn = 1000
    grid = (N_NODES // bn,)
    half_spec = pl.BlockSpec((bn, DH), lambda i: (i, 0))
    row_spec = pl.BlockSpec((bn, D), lambda i: (i, 0))
    w_spec = pl.BlockSpec((D, D), lambda i: (0, 0))
    flat_aggs = [a for pair in aggs for a in pair]
    return pl.pallas_call(
        _tc_body,
        grid=grid,
        in_specs=[half_spec] * 8 + [row_spec] + [w_spec] * 5,
        out_specs=pl.BlockSpec((bn, 4 * D), lambda i: (i, 0)),
        out_shape=jax.ShapeDtypeStruct((N_NODES, 4 * D), jnp.float32),
    )(*flat_aggs, node_features, W_pre_s, W_pre_v, W_post_s, W_post_v, W_sc)


def kernel(node_features, positions, senders, receivers,
           W_pre_s, W_pre_v, W_post_s, W_post_v, W_sc):
    aggs = _sc_aggregate(node_features, positions, senders, receivers)
    return _tc_update(aggs, node_features,
                      W_pre_s, W_pre_v, W_post_s, W_post_v, W_sc)


# speedup vs baseline: 14.5774x; 1.4802x over previous
import jax
import jax.numpy as jnp
from jax import lax
from jax.experimental import pallas as pl
from jax.experimental.pallas import tpu as pltpu
from jax.experimental.pallas import tpu_sc as plsc

N_NODES = 10000
N_EDGES = 320000
D = 128
DH = 64

N_TILES = 16
EPT = 20480
E_PAD = EPT * N_TILES
BLK = 128
NBLK = EPT // BLK
GBLK = 16
GEDGE = GBLK * BLK
NGRP = NBLK // GBLK
N_PAD = N_NODES + 8
RPT = 632
RPT_LAST = N_NODES - 15 * RPT

_SQRT3 = 3.0 ** 0.5


def _rsqrt(x):
    i = lax.bitcast_convert_type(x, jnp.int32)
    i = jnp.int32(0x5F3759DF) - (i >> 1)
    y = lax.bitcast_convert_type(i, jnp.float32)
    for _ in range(3):
        y = y * (1.5 - 0.5 * x * y * y)
    return y


def _sc_body(nf0_hbm, nf1_hbm, px_hbm, py_hbm, pz_hbm, snd_hbm, rcv2_hbm,
             o00, o10, o01, o11, o20, o30, o21, o31,
             px_v, py_v, pz_v, sidxA, sidxB, ridxA, ridxB, wbuf,
             rows0, rows1, wrow0, wrow1, zbuf, acc,
             isem0, isem1, gsem0, gsem1, ssem0, ssem1):
    cid = lax.axis_index("c")
    sid = lax.axis_index("s")
    sweeps = ((0, o00, o10), (1, o01, o11), (0, o20, o30), (1, o21, o31))
    rows = (rows0, rows1)
    wrow = (wrow0, wrow1)
    sidx = (sidxA, sidxB)
    ridx = (ridxA, ridxB)
    isem = (isem0, isem1)
    gsem = (gsem0, gsem1)
    ssem = (ssem0, ssem1)

    pltpu.sync_copy(px_hbm, px_v)
    pltpu.sync_copy(py_hbm, py_v)
    pltpu.sync_copy(pz_hbm, pz_v)

    def _zero_row(i, _):
        for j in range(DH // 16):
            zbuf[i, pl.ds(j * 16, 16)] = jnp.zeros((16,), jnp.float32)
        return 0
    lax.fori_loop(0, 64, _zero_row, 0)

    zeros16i = jnp.zeros((16,), jnp.int32)
    ones16 = jnp.ones((16,), jnp.float32)
    r0 = pl.multiple_of(sid * RPT, 8)

    def _fire_idx(g, q):
        e0 = sid * EPT + g * GEDGE
        pltpu.async_copy(snd_hbm.at[pl.ds(e0, GEDGE)], sidx[q], isem[q])
        pltpu.async_copy(rcv2_hbm.at[pl.ds(sid * NBLK + g * GBLK, GBLK)],
                         ridx[q], isem[q])

    def _drain_idx(q):
        pltpu.make_async_copy(
            snd_hbm.at[pl.ds(0, GEDGE)], sidx[q], isem[q]).wait()
        pltpu.make_async_copy(
            rcv2_hbm.at[pl.ds(0, GBLK)], ridx[q], isem[q]).wait()

    def _fire_gather(nf_hbm, q, b_in, p):
        return pltpu.async_copy(
            nf_hbm.at[sidx[q].at[pl.ds(b_in * BLK, BLK)]], rows[p], gsem[p])

    def _drain_gather(nf_hbm, p):
        pltpu.make_async_copy(nf_hbm.at[pl.ds(0, BLK)], rows[p], gsem[p]).wait()

    def _drain_scatter(nf_hbm, p):
        pltpu.make_async_copy(nf_hbm.at[pl.ds(0, BLK)], wrow[p], ssem[p]).wait()

    for swp, (half, out_a, out_b) in enumerate(sweeps):
        nf_hbm = nf0_hbm if half == 0 else nf1_hbm

        for j in range(8):
            pltpu.sync_copy(zbuf, acc.at[pl.ds(r0 + j * 64, 64)])

        @pl.when(sid < N_TILES - 1)
        def _():
            pltpu.sync_copy(zbuf, acc.at[pl.ds(r0 + 512, 64)])
            pltpu.sync_copy(zbuf.at[pl.ds(0, RPT - 576)],
                            acc.at[pl.ds(r0 + 576, RPT - 576)])

        @pl.when(sid == N_TILES - 1)
        def _():
            pltpu.sync_copy(zbuf.at[pl.ds(0, N_PAD - 15 * RPT - 512)],
                            acc.at[pl.ds(15 * RPT + 512, N_PAD - 15 * RPT - 512)])

        _fire_idx(0, 0)
        plsc.subcore_barrier()

        def _group_pair(g2, _):
            for q in range(2):
                g = 2 * g2 + q
                if q == 0:
                    @pl.when(g2 > 0)
                    def _():
                        _drain_scatter(nf_hbm, 0)
                        _drain_scatter(nf_hbm, 1)
                else:
                    _drain_scatter(nf_hbm, 0)
                    _drain_scatter(nf_hbm, 1)
                _drain_idx(q)
                if q == 0:
                    _fire_idx(g + 1, 1)
                else:
                    @pl.when(g2 < NGRP // 2 - 1)
                    def _():
                        _fire_idx(g + 1, 0)
                _fire_gather(nf_hbm, q, 0, 0)
                _fire_gather(nf_hbm, q, 1, 1)

                def _pair(it, _, q=q):
                    for p in range(2):
                        b_in = 2 * it + p
                        _drain_gather(nf_hbm, p)

                        for i in range(BLK // 16):
                            s16 = sidx[q][pl.ds(b_in * BLK + i * 16, 16)]
                            r16 = ridx[q][b_in, pl.ds(i * 16, 16)]
                            sx = plsc.load_gather(px_v, [s16])
                            sy = plsc.load_gather(py_v, [s16])
                            sz = plsc.load_gather(pz_v, [s16])
                            rx = plsc.load_gather(px_v, [r16])
                            ry = plsc.load_gather(py_v, [r16])
                            rz = plsc.load_gather(pz_v, [r16])
                            vx, vy, vz = rx - sx, ry - sy, rz - sz
                            rinv = _rsqrt(
                                vx * vx + vy * vy + vz * vz + 1e-12) * _SQRT3
                            if swp < 2:
                                w16 = jnp.where(cid == 0, ones16, vx * rinv)
                            else:
                                w16 = jnp.where(cid == 0, vy * rinv, vz * rinv)
                            wbuf[p, pl.ds(i * 16, 16)] = w16

                        @pl.when(b_in >= 2)
                        def _(p=p):
                            _drain_scatter(nf_hbm, p)

                        def _mul(e, _, p=p):
                            wsp = plsc.load_gather(
                                wbuf, [zeros16i + p, zeros16i + e])
                            for j in range(DH // 16):
                                wrow[p][e, pl.ds(j * 16, 16)] = (
                                    wsp * rows[p][e, pl.ds(j * 16, 16)])
                            return 0
                        lax.fori_loop(0, BLK, _mul, 0)

                        pltpu.async_copy(wrow[p], acc.at[ridx[q].at[b_in]],
                                         ssem[p], add=True)

                        @pl.when(b_in + 2 < GBLK)
                        def _(q=q, p=p, b_in=b_in):
                            _fire_gather(nf_hbm, q, b_in + 2, p)
                    return 0

                lax.fori_loop(0, GBLK // 2, _pair, 0)
            return 0

        lax.fori_loop(0, NGRP // 2, _group_pair, 0)
        _drain_scatter(nf_hbm, 0)
        _drain_scatter(nf_hbm, 1)

        plsc.subcore_barrier()

        for this_cid, out_x in ((0, out_a), (1, out_b)):
            @pl.when((cid == this_cid) & (sid < N_TILES - 1))
            def _(out_x=out_x):
                pltpu.sync_copy(acc.at[pl.ds(r0, RPT)],
                                out_x.at[pl.ds(r0, RPT)])

            @pl.when((cid == this_cid) & (sid == N_TILES - 1))
            def _(out_x=out_x):
                pltpu.sync_copy(acc.at[pl.ds(15 * RPT, RPT_LAST)],
                                out_x.at[pl.ds(15 * RPT, RPT_LAST)])


def _sc_aggregate(node_features, positions, senders, receivers):
    pad = E_PAD - N_EDGES
    snd_p = jnp.concatenate([senders, jnp.zeros((pad,), jnp.int32)])
    rcv_p = jnp.concatenate([receivers, jnp.full((pad,), N_NODES, jnp.int32)])
    rcv2 = rcv_p.reshape(E_PAD // BLK, BLK)

    nf0 = node_features[:, :DH]
    nf1 = node_features[:, DH:]
    px, py, pz = positions[:, 0], positions[:, 1], positions[:, 2]
    mesh = plsc.VectorSubcoreMesh(core_axis_name="c", subcore_axis_name="s")
    f32 = jnp.float32
    agg_shape = jax.ShapeDtypeStruct((N_NODES, DH), f32)
    fn = pl.kernel(
        _sc_body,
        mesh=mesh,
        compiler_params=pltpu.CompilerParams(
            needs_layout_passes=False, use_tc_tiling_on_sc=False),
        out_type=tuple(agg_shape for _ in range(8)),
        scratch_types=[
            pltpu.VMEM((N_NODES,), f32),
            pltpu.VMEM((N_NODES,), f32),
            pltpu.VMEM((N_NODES,), f32),
            pltpu.VMEM((GEDGE,), jnp.int32),
            pltpu.VMEM((GEDGE,), jnp.int32),
            pltpu.VMEM((GBLK, BLK), jnp.int32),
            pltpu.VMEM((GBLK, BLK), jnp.int32),
            pltpu.VMEM((2, BLK), f32),
            pltpu.VMEM((BLK, DH), f32),
            pltpu.VMEM((BLK, DH), f32),
            pltpu.VMEM((BLK, DH), f32),
            pltpu.VMEM((BLK, DH), f32),
            pltpu.VMEM((64, DH), f32),
            pltpu.VMEM_SHARED((N_PAD, DH), f32),
            pltpu.SemaphoreType.DMA,
            pltpu.SemaphoreType.DMA,
            pltpu.SemaphoreType.DMA,
            pltpu.SemaphoreType.DMA,
            pltpu.SemaphoreType.DMA,
            pltpu.SemaphoreType.DMA,
        ],
    )
    o00, o10, o01, o11, o20, o30, o21, o31 = fn(
        nf0, nf1, px, py, pz, snd_p, rcv2)
    return ((o00, o01), (o10, o11), (o20, o21), (o30, o31))


def _tc_body(a0l_ref, a0h_ref, a1l_ref, a1h_ref, a2l_ref, a2h_ref,
             a3l_ref, a3h_ref, nf_ref,
             wps_ref, wpv_ref, wos_ref, wov_ref, wsc_ref, out_ref):
    inv = 1.0 / (D ** 0.5)
    den = 1.0 / 32.0
    f32 = jnp.float32

    def matmul_split(lo, hi, w_ref):
        return (jnp.dot(lo, w_ref[0:DH, :], preferred_element_type=f32)
                + jnp.dot(hi, w_ref[DH:D, :], preferred_element_type=f32))

    s1 = matmul_split(a0l_ref[...] * den, a0h_ref[...] * den, wps_ref) * inv
    s1 = s1 * jax.nn.sigmoid(s1)
    s2 = jnp.dot(s1, wos_ref[...], preferred_element_type=f32) * inv
    sc = jnp.dot(nf_ref[...], wsc_ref[...], preferred_element_type=f32) * inv
    out_ref[:, 0:D] = sc + s2

    rows = lax.broadcasted_iota(jnp.int32, (D, 3 * D), 0)
    cols = lax.broadcasted_iota(jnp.int32, (D, 3 * D), 1)
    outv = jnp.zeros((out_ref.shape[0], 3 * D), f32)
    for i, (lo_ref, hi_ref) in enumerate(
            ((a1l_ref, a1h_ref), (a2l_ref, a2h_ref), (a3l_ref, a3h_ref))):
        v1 = matmul_split(lo_ref[...] * den, hi_ref[...] * den, wpv_ref) * inv
        v2 = jnp.dot(v1, wov_ref[...], preferred_element_type=f32) * inv
        perm = (cols == 3 * rows + i).astype(f32)
        outv = outv + jnp.dot(v2, perm, preferred_element_type=f32)
    out_ref[:, D:4 * D] = outv


def _tc_update(aggs, node_features, W_pre_s, W_pre_v, W_post_s, W_post_v, W_sc):
    bn = 1000
    grid = (N_NODES // bn,)
    half_spec = pl.BlockSpec((bn, DH), lambda i: (i, 0))
    row_spec = pl.BlockSpec((bn, D), lambda i: (i, 0))
    w_spec = pl.BlockSpec((D, D), lambda i: (0, 0))
    flat_aggs = [a for pair in aggs for a in pair]
    return pl.pallas_call(
        _tc_body,
        grid=grid,
        in_specs=[half_spec] * 8 + [row_spec] + [w_spec] * 5,
        out_specs=pl.BlockSpec((bn, 4 * D), lambda i: (i, 0)),
        out_shape=jax.ShapeDtypeStruct((N_NODES, 4 * D), jnp.float32),
    )(*flat_aggs, node_features, W_pre_s, W_pre_v, W_post_s, W_post_v, W_sc)


def kernel(node_features, positions, senders, receivers,
           W_pre_s, W_pre_v, W_post_s, W_post_v, W_sc):
    aggs = _sc_aggregate(node_features, positions, senders, receivers)
    return _tc_update(aggs, node_features,
                      W_pre_s, W_pre_v, W_post_s, W_post_v, W_sc)


# speedup vs baseline: 20.9786x; 1.4391x over previous
import jax
import jax.numpy as jnp
from jax import lax
from jax.experimental import pallas as pl
from jax.experimental.pallas import tpu as pltpu
from jax.experimental.pallas import tpu_sc as plsc

N_NODES = 10000
N_EDGES = 320000
D = 128
DH = 64

N_TILES = 16
EPT = 20480
E_PAD = EPT * N_TILES
BLK = 128
NBLK = EPT // BLK
GBLK = 16
GEDGE = GBLK * BLK
NGRP = NBLK // GBLK
N_PAD = N_NODES + 8
RPT = 632
RPT_LAST = N_NODES - 15 * RPT

_SQRT3 = 3.0 ** 0.5


def _rsqrt(x):
    i = lax.bitcast_convert_type(x, jnp.int32)
    i = jnp.int32(0x5F3759DF) - (i >> 1)
    y = lax.bitcast_convert_type(i, jnp.float32)
    for _ in range(3):
        y = y * (1.5 - 0.5 * x * y * y)
    return y


def _sc_body(nf0_hbm, nf1_hbm, px_hbm, py_hbm, pz_hbm, snd_hbm, rcv2_hbm,
             o00, o10, o01, o11, o20, o30, o21, o31,
             px_v, py_v, pz_v, sidxA, sidxB, ridxA, ridxB, wbuf,
             rows0, rows1, wrow0, wrow1, zbuf, acc,
             isem0, isem1, gsem0, gsem1, ssem0, ssem1):
    cid = lax.axis_index("c")
    sid = lax.axis_index("s")
    sweeps = ((0, o00, o10), (1, o01, o11), (0, o20, o30), (1, o21, o31))
    rows = (rows0, rows1)
    wrow = (wrow0, wrow1)
    sidx = (sidxA, sidxB)
    ridx = (ridxA, ridxB)
    isem = (isem0, isem1)
    gsem = (gsem0, gsem1)
    ssem = (ssem0, ssem1)

    pltpu.sync_copy(px_hbm, px_v)
    pltpu.sync_copy(py_hbm, py_v)
    pltpu.sync_copy(pz_hbm, pz_v)

    def _zero_row(i, _):
        for j in range(DH // 16):
            zbuf[i, pl.ds(j * 16, 16)] = jnp.zeros((16,), jnp.float32)
        return 0
    lax.fori_loop(0, 64, _zero_row, 0)

    zeros16i = jnp.zeros((16,), jnp.int32)
    ones16 = jnp.ones((16,), jnp.float32)
    r0 = pl.multiple_of(sid * RPT, 8)

    def _fire_idx(g, q):
        e0 = sid * EPT + g * GEDGE
        pltpu.async_copy(snd_hbm.at[pl.ds(e0, GEDGE)], sidx[q], isem[q])
        pltpu.async_copy(rcv2_hbm.at[pl.ds(sid * NBLK + g * GBLK, GBLK)],
                         ridx[q], isem[q])

    def _drain_idx(q):
        pltpu.make_async_copy(
            snd_hbm.at[pl.ds(0, GEDGE)], sidx[q], isem[q]).wait()
        pltpu.make_async_copy(
            rcv2_hbm.at[pl.ds(0, GBLK)], ridx[q], isem[q]).wait()

    def _fire_gather(nf_hbm, q, b_in, p):
        return pltpu.async_copy(
            nf_hbm.at[sidx[q].at[pl.ds(b_in * BLK, BLK)]], rows[p], gsem[p])

    def _drain_gather(nf_hbm, p):
        pltpu.make_async_copy(nf_hbm.at[pl.ds(0, BLK)], rows[p], gsem[p]).wait()

    def _drain_scatter(nf_hbm, p):
        pltpu.make_async_copy(nf_hbm.at[pl.ds(0, BLK)], wrow[p], ssem[p]).wait()

    for swp, (half, out_a, out_b) in enumerate(sweeps):
        nf_hbm = nf0_hbm if half == 0 else nf1_hbm

        for j in range(8):
            pltpu.sync_copy(zbuf, acc.at[pl.ds(r0 + j * 64, 64)])

        @pl.when(sid < N_TILES - 1)
        def _():
            pltpu.sync_copy(zbuf, acc.at[pl.ds(r0 + 512, 64)])
            pltpu.sync_copy(zbuf.at[pl.ds(0, RPT - 576)],
                            acc.at[pl.ds(r0 + 576, RPT - 576)])

        @pl.when(sid == N_TILES - 1)
        def _():
            pltpu.sync_copy(zbuf.at[pl.ds(0, N_PAD - 15 * RPT - 512)],
                            acc.at[pl.ds(15 * RPT + 512, N_PAD - 15 * RPT - 512)])

        _fire_idx(0, 0)
        plsc.subcore_barrier()

        def _group_pair(g2, _):
            for q in range(2):
                g = 2 * g2 + q
                if q == 0:
                    @pl.when(g2 > 0)
                    def _():
                        _drain_scatter(nf_hbm, 0)
                        _drain_scatter(nf_hbm, 1)
                else:
                    _drain_scatter(nf_hbm, 0)
                    _drain_scatter(nf_hbm, 1)
                _drain_idx(q)
                if q == 0:
                    _fire_idx(g + 1, 1)
                else:
                    @pl.when(g2 < NGRP // 2 - 1)
                    def _():
                        _fire_idx(g + 1, 0)
                _fire_gather(nf_hbm, q, 0, 0)
                _fire_gather(nf_hbm, q, 1, 1)

                def _pair(it, _, q=q):
                    for p in range(2):
                        b_in = 2 * it + p
                        _drain_gather(nf_hbm, p)

                        for i in range(BLK // 16):
                            s16 = sidx[q][pl.ds(b_in * BLK + i * 16, 16)]
                            r16 = ridx[q][b_in, pl.ds(i * 16, 16)]
                            sx = plsc.load_gather(px_v, [s16])
                            sy = plsc.load_gather(py_v, [s16])
                            sz = plsc.load_gather(pz_v, [s16])
                            rx = plsc.load_gather(px_v, [r16])
                            ry = plsc.load_gather(py_v, [r16])
                            rz = plsc.load_gather(pz_v, [r16])
                            vx, vy, vz = rx - sx, ry - sy, rz - sz
                            rinv = _rsqrt(
                                vx * vx + vy * vy + vz * vz + 1e-12) * _SQRT3
                            if swp < 2:
                                w16 = jnp.where(cid == 0, ones16, vx * rinv)
                            else:
                                w16 = jnp.where(cid == 0, vy * rinv, vz * rinv)
                            wbuf[p, pl.ds(i * 16, 16)] = w16

                        @pl.when(b_in >= 2)
                        def _(p=p):
                            _drain_scatter(nf_hbm, p)

                        @plsc.parallel_loop(0, BLK, 1, unroll=8)
                        def _mul(e, p=p):
                            wsp = plsc.load_gather(
                                wbuf, [zeros16i + p, zeros16i + e])
                            for j in range(DH // 16):
                                wrow[p][e, pl.ds(j * 16, 16)] = (
                                    wsp * rows[p][e, pl.ds(j * 16, 16)])

                        pltpu.async_copy(wrow[p], acc.at[ridx[q].at[b_in]],
                                         ssem[p], add=True)

                        @pl.when(b_in + 2 < GBLK)
                        def _(q=q, p=p, b_in=b_in):
                            _fire_gather(nf_hbm, q, b_in + 2, p)
                    return 0

                lax.fori_loop(0, GBLK // 2, _pair, 0)
            return 0

        lax.fori_loop(0, NGRP // 2, _group_pair, 0)
        _drain_scatter(nf_hbm, 0)
        _drain_scatter(nf_hbm, 1)

        plsc.subcore_barrier()

        for this_cid, out_x in ((0, out_a), (1, out_b)):
            @pl.when((cid == this_cid) & (sid < N_TILES - 1))
            def _(out_x=out_x):
                pltpu.sync_copy(acc.at[pl.ds(r0, RPT)],
                                out_x.at[pl.ds(r0, RPT)])

            @pl.when((cid == this_cid) & (sid == N_TILES - 1))
            def _(out_x=out_x):
                pltpu.sync_copy(acc.at[pl.ds(15 * RPT, RPT_LAST)],
                                out_x.at[pl.ds(15 * RPT, RPT_LAST)])


def _sc_aggregate(node_features, positions, senders, receivers):
    pad = E_PAD - N_EDGES
    snd_p = jnp.concatenate([senders, jnp.zeros((pad,), jnp.int32)])
    rcv_p = jnp.concatenate([receivers, jnp.full((pad,), N_NODES, jnp.int32)])
    rcv2 = rcv_p.reshape(E_PAD // BLK, BLK)

    nf0 = node_features[:, :DH]
    nf1 = node_features[:, DH:]
    px, py, pz = positions[:, 0], positions[:, 1], positions[:, 2]
    mesh = plsc.VectorSubcoreMesh(core_axis_name="c", subcore_axis_name="s")
    f32 = jnp.float32
    agg_shape = jax.ShapeDtypeStruct((N_NODES, DH), f32)
    fn = pl.kernel(
        _sc_body,
        mesh=mesh,
        compiler_params=pltpu.CompilerParams(
            needs_layout_passes=False, use_tc_tiling_on_sc=False),
        out_type=tuple(agg_shape for _ in range(8)),
        scratch_types=[
            pltpu.VMEM((N_NODES,), f32),
            pltpu.VMEM((N_NODES,), f32),
            pltpu.VMEM((N_NODES,), f32),
            pltpu.VMEM((GEDGE,), jnp.int32),
            pltpu.VMEM((GEDGE,), jnp.int32),
            pltpu.VMEM((GBLK, BLK), jnp.int32),
            pltpu.VMEM((GBLK, BLK), jnp.int32),
            pltpu.VMEM((2, BLK), f32),
            pltpu.VMEM((BLK, DH), f32),
            pltpu.VMEM((BLK, DH), f32),
            pltpu.VMEM((BLK, DH), f32),
            pltpu.VMEM((BLK, DH), f32),
            pltpu.VMEM((64, DH), f32),
            pltpu.VMEM_SHARED((N_PAD, DH), f32),
            pltpu.SemaphoreType.DMA,
            pltpu.SemaphoreType.DMA,
            pltpu.SemaphoreType.DMA,
            pltpu.SemaphoreType.DMA,
            pltpu.SemaphoreType.DMA,
            pltpu.SemaphoreType.DMA,
        ],
    )
    o00, o10, o01, o11, o20, o30, o21, o31 = fn(
        nf0, nf1, px, py, pz, snd_p, rcv2)
    return ((o00, o01), (o10, o11), (o20, o21), (o30, o31))


def _tc_body(a0l_ref, a0h_ref, a1l_ref, a1h_ref, a2l_ref, a2h_ref,
             a3l_ref, a3h_ref, nf_ref,
             wps_ref, wpv_ref, wos_ref, wov_ref, wsc_ref, out_ref):
    inv = 1.0 / (D ** 0.5)
    den = 1.0 / 32.0
    f32 = jnp.float32

    def matmul_split(lo, hi, w_ref):
        return (jnp.dot(lo, w_ref[0:DH, :], preferred_element_type=f32)
                + jnp.dot(hi, w_ref[DH:D, :], preferred_element_type=f32))

    s1 = matmul_split(a0l_ref[...] * den, a0h_ref[...] * den, wps_ref) * inv
    s1 = s1 * jax.nn.sigmoid(s1)
    s2 = jnp.dot(s1, wos_ref[...], preferred_element_type=f32) * inv
    sc = jnp.dot(nf_ref[...], wsc_ref[...], preferred_element_type=f32) * inv
    out_ref[:, 0:D] = sc + s2

    rows = lax.broadcasted_iota(jnp.int32, (D, 3 * D), 0)
    cols = lax.broadcasted_iota(jnp.int32, (D, 3 * D), 1)
    outv = jnp.zeros((out_ref.shape[0], 3 * D), f32)
    for i, (lo_ref, hi_ref) in enumerate(
            ((a1l_ref, a1h_ref), (a2l_ref, a2h_ref), (a3l_ref, a3h_ref))):
        v1 = matmul_split(lo_ref[...] * den, hi_ref[...] * den, wpv_ref) * inv
        v2 = jnp.dot(v1, wov_ref[...], preferred_element_type=f32) * inv
        perm = (cols == 3 * rows + i).astype(f32)
        outv = outv + jnp.dot(v2, perm, preferred_element_type=f32)
    out_ref[:, D:4 * D] = outv


def _tc_update(aggs, node_features, W_pre_s, W_pre_v, W_post_s, W_post_v, W_sc):
    bn = 1000
    grid = (N_NODES // bn,)
    half_spec = pl.BlockSpec((bn, DH), lambda i: (i, 0))
    row_spec = pl.BlockSpec((bn, D), lambda i: (i, 0))
    w_spec = pl.BlockSpec((D, D), lambda i: (0, 0))
    flat_aggs = [a for pair in aggs for a in pair]
    return pl.pallas_call(
        _tc_body,
        grid=grid,
        in_specs=[half_spec] * 8 + [row_spec] + [w_spec] * 5,
        out_specs=pl.BlockSpec((bn, 4 * D), lambda i: (i, 0)),
        out_shape=jax.ShapeDtypeStruct((N_NODES, 4 * D), jnp.float32),
    )(*flat_aggs, node_features, W_pre_s, W_pre_v, W_post_s, W_post_v, W_sc)


def kernel(node_features, positions, senders, receivers,
           W_pre_s, W_pre_v, W_post_s, W_post_v, W_sc):
    aggs = _sc_aggregate(node_features, positions, senders, receivers)
    return _tc_update(aggs, node_features,
                      W_pre_s, W_pre_v, W_post_s, W_post_v, W_sc)


# speedup vs baseline: 21.6364x; 1.0314x over previous
import jax
import jax.numpy as jnp
from jax import lax
from jax.experimental import pallas as pl
from jax.experimental.pallas import tpu as pltpu
from jax.experimental.pallas import tpu_sc as plsc

N_NODES = 10000
N_EDGES = 320000
D = 128
DH = 64

N_TILES = 16
EPT = 20480
E_PAD = EPT * N_TILES
BLK = 128
NBLK = EPT // BLK
GBLK = 16
GEDGE = GBLK * BLK
NGRP = NBLK // GBLK
N_PAD = N_NODES + 8
RPT = 632
RPT_LAST = N_NODES - 15 * RPT

_SQRT3 = 3.0 ** 0.5


def _rsqrt(x):
    i = lax.bitcast_convert_type(x, jnp.int32)
    i = jnp.int32(0x5F3759DF) - (i >> 1)
    y = lax.bitcast_convert_type(i, jnp.float32)
    for _ in range(3):
        y = y * (1.5 - 0.5 * x * y * y)
    return y


def _sc_body(nf0_hbm, nf1_hbm, px_hbm, py_hbm, pz_hbm, snd_hbm, rcv2_hbm,
             o00, o10, o01, o11, o20, o30, o21, o31,
             px_v, py_v, pz_v, sidxA, sidxB, ridxA, ridxB, wbuf,
             rows0, rows1, wrow0, wrow1, zbuf, acc,
             isem0, isem1, gsem0, gsem1, ssem0, ssem1):
    cid = lax.axis_index("c")
    sid = lax.axis_index("s")
    sweeps = ((0, o00, o10), (1, o01, o11), (0, o20, o30), (1, o21, o31))
    rows = (rows0, rows1)
    wrow = (wrow0, wrow1)
    sidx = (sidxA, sidxB)
    ridx = (ridxA, ridxB)
    isem = (isem0, isem1)
    gsem = (gsem0, gsem1)
    ssem = (ssem0, ssem1)

    pltpu.sync_copy(px_hbm, px_v)
    pltpu.sync_copy(py_hbm, py_v)
    pltpu.sync_copy(pz_hbm, pz_v)

    def _zero_row(i, _):
        for j in range(DH // 16):
            zbuf[i, pl.ds(j * 16, 16)] = jnp.zeros((16,), jnp.float32)
        return 0
    lax.fori_loop(0, 64, _zero_row, 0)

    zeros16i = jnp.zeros((16,), jnp.int32)
    ones16 = jnp.ones((16,), jnp.float32)
    r0 = pl.multiple_of(sid * RPT, 8)

    def _fire_idx(g, q):
        e0 = sid * EPT + g * GEDGE
        pltpu.async_copy(snd_hbm.at[pl.ds(e0, GEDGE)], sidx[q], isem[q])
        pltpu.async_copy(rcv2_hbm.at[pl.ds(sid * NBLK + g * GBLK, GBLK)],
                         ridx[q], isem[q])

    def _drain_idx(q):
        pltpu.make_async_copy(
            snd_hbm.at[pl.ds(0, GEDGE)], sidx[q], isem[q]).wait()
        pltpu.make_async_copy(
            rcv2_hbm.at[pl.ds(0, GBLK)], ridx[q], isem[q]).wait()

    def _fire_gather(nf_hbm, q, b_in, p):
        return pltpu.async_copy(
            nf_hbm.at[sidx[q].at[pl.ds(b_in * BLK, BLK)]], rows[p], gsem[p])

    def _drain_gather(nf_hbm, p):
        pltpu.make_async_copy(nf_hbm.at[pl.ds(0, BLK)], rows[p], gsem[p]).wait()

    def _drain_scatter(nf_hbm, p):
        pltpu.make_async_copy(nf_hbm.at[pl.ds(0, BLK)], wrow[p], ssem[p]).wait()

    for swp, (half, out_a, out_b) in enumerate(sweeps):
        nf_hbm = nf0_hbm if half == 0 else nf1_hbm

        for j in range(8):
            pltpu.sync_copy(zbuf, acc.at[pl.ds(r0 + j * 64, 64)])

        @pl.when(sid < N_TILES - 1)
        def _():
            pltpu.sync_copy(zbuf, acc.at[pl.ds(r0 + 512, 64)])
            pltpu.sync_copy(zbuf.at[pl.ds(0, RPT - 576)],
                            acc.at[pl.ds(r0 + 576, RPT - 576)])

        @pl.when(sid == N_TILES - 1)
        def _():
            pltpu.sync_copy(zbuf.at[pl.ds(0, N_PAD - 15 * RPT - 512)],
                            acc.at[pl.ds(15 * RPT + 512, N_PAD - 15 * RPT - 512)])

        _fire_idx(0, 0)
        plsc.subcore_barrier()

        def _group_pair(g2, _):
            for q in range(2):
                g = 2 * g2 + q
                if q == 0:
                    @pl.when(g2 > 0)
                    def _():
                        _drain_scatter(nf_hbm, 0)
                        _drain_scatter(nf_hbm, 1)
                else:
                    _drain_scatter(nf_hbm, 0)
                    _drain_scatter(nf_hbm, 1)
                _drain_idx(q)
                if q == 0:
                    _fire_idx(g + 1, 1)
                else:
                    @pl.when(g2 < NGRP // 2 - 1)
                    def _():
                        _fire_idx(g + 1, 0)
                _fire_gather(nf_hbm, q, 0, 0)
                _fire_gather(nf_hbm, q, 1, 1)

                def _pair(it, _, q=q):
                    for p in range(2):
                        b_in = 2 * it + p
                        _drain_gather(nf_hbm, p)

                        def _weights(sel):
                            @plsc.parallel_loop(0, BLK // 16, 1, unroll=4)
                            def _w(i, q=q, b_in=b_in, p=p):
                                s16 = sidx[q][pl.ds(b_in * BLK + i * 16, 16)]
                                r16 = ridx[q][b_in, pl.ds(i * 16, 16)]
                                sx = plsc.load_gather(px_v, [s16])
                                sy = plsc.load_gather(py_v, [s16])
                                sz = plsc.load_gather(pz_v, [s16])
                                rx = plsc.load_gather(px_v, [r16])
                                ry = plsc.load_gather(py_v, [r16])
                                rz = plsc.load_gather(pz_v, [r16])
                                vx, vy, vz = rx - sx, ry - sy, rz - sz
                                rinv = _rsqrt(
                                    vx * vx + vy * vy + vz * vz + 1e-12) * _SQRT3
                                wbuf[p, pl.ds(i * 16, 16)] = sel(vx, vy, vz, rinv)

                        if swp < 2:
                            @pl.when(cid == 0)
                            def _(p=p):
                                @plsc.parallel_loop(0, BLK // 16, 1, unroll=4)
                                def _w1(i, p=p):
                                    wbuf[p, pl.ds(i * 16, 16)] = ones16

                            @pl.when(cid == 1)
                            def _():
                                _weights(lambda vx, vy, vz, rinv: vx * rinv)
                        else:
                            _weights(lambda vx, vy, vz, rinv: jnp.where(
                                cid == 0, vy * rinv, vz * rinv))

                        @pl.when(b_in >= 2)
                        def _(p=p):
                            _drain_scatter(nf_hbm, p)

                        @plsc.parallel_loop(0, BLK, 1, unroll=8)
                        def _mul(e, p=p):
                            wsp = plsc.load_gather(
                                wbuf, [zeros16i + p, zeros16i + e])
                            for j in range(DH // 16):
                                wrow[p][e, pl.ds(j * 16, 16)] = (
                                    wsp * rows[p][e, pl.ds(j * 16, 16)])

                        pltpu.async_copy(wrow[p], acc.at[ridx[q].at[b_in]],
                                         ssem[p], add=True)

                        @pl.when(b_in + 2 < GBLK)
                        def _(q=q, p=p, b_in=b_in):
                            _fire_gather(nf_hbm, q, b_in + 2, p)
                    return 0

                lax.fori_loop(0, GBLK // 2, _pair, 0)
            return 0

        lax.fori_loop(0, NGRP // 2, _group_pair, 0)
        _drain_scatter(nf_hbm, 0)
        _drain_scatter(nf_hbm, 1)

        plsc.subcore_barrier()

        for this_cid, out_x in ((0, out_a), (1, out_b)):
            @pl.when((cid == this_cid) & (sid < N_TILES - 1))
            def _(out_x=out_x):
                pltpu.sync_copy(acc.at[pl.ds(r0, RPT)],
                                out_x.at[pl.ds(r0, RPT)])

            @pl.when((cid == this_cid) & (sid == N_TILES - 1))
            def _(out_x=out_x):
                pltpu.sync_copy(acc.at[pl.ds(15 * RPT, RPT_LAST)],
                                out_x.at[pl.ds(15 * RPT, RPT_LAST)])


def _sc_aggregate(node_features, positions, senders, receivers):
    pad = E_PAD - N_EDGES
    snd_p = jnp.concatenate([senders, jnp.zeros((pad,), jnp.int32)])
    rcv_p = jnp.concatenate([receivers, jnp.full((pad,), N_NODES, jnp.int32)])
    rcv2 = rcv_p.reshape(E_PAD // BLK, BLK)

    nf0 = node_features[:, :DH]
    nf1 = node_features[:, DH:]
    px, py, pz = positions[:, 0], positions[:, 1], positions[:, 2]
    mesh = plsc.VectorSubcoreMesh(core_axis_name="c", subcore_axis_name="s")
    f32 = jnp.float32
    agg_shape = jax.ShapeDtypeStruct((N_NODES, DH), f32)
    fn = pl.kernel(
        _sc_body,
        mesh=mesh,
        compiler_params=pltpu.CompilerParams(
            needs_layout_passes=False, use_tc_tiling_on_sc=False),
        out_type=tuple(agg_shape for _ in range(8)),
        scratch_types=[
            pltpu.VMEM((N_NODES,), f32),
            pltpu.VMEM((N_NODES,), f32),
            pltpu.VMEM((N_NODES,), f32),
            pltpu.VMEM((GEDGE,), jnp.int32),
            pltpu.VMEM((GEDGE,), jnp.int32),
            pltpu.VMEM((GBLK, BLK), jnp.int32),
            pltpu.VMEM((GBLK, BLK), jnp.int32),
            pltpu.VMEM((2, BLK), f32),
            pltpu.VMEM((BLK, DH), f32),
            pltpu.VMEM((BLK, DH), f32),
            pltpu.VMEM((BLK, DH), f32),
            pltpu.VMEM((BLK, DH), f32),
            pltpu.VMEM((64, DH), f32),
            pltpu.VMEM_SHARED((N_PAD, DH), f32),
            pltpu.SemaphoreType.DMA,
            pltpu.SemaphoreType.DMA,
            pltpu.SemaphoreType.DMA,
            pltpu.SemaphoreType.DMA,
            pltpu.SemaphoreType.DMA,
            pltpu.SemaphoreType.DMA,
        ],
    )
    o00, o10, o01, o11, o20, o30, o21, o31 = fn(
        nf0, nf1, px, py, pz, snd_p, rcv2)
    return ((o00, o01), (o10, o11), (o20, o21), (o30, o31))


def _tc_body(a0l_ref, a0h_ref, a1l_ref, a1h_ref, a2l_ref, a2h_ref,
             a3l_ref, a3h_ref, nf_ref,
             wps_ref, wpv_ref, wos_ref, wov_ref, wsc_ref, out_ref):
    inv = 1.0 / (D ** 0.5)
    den = 1.0 / 32.0
    f32 = jnp.float32

    def matmul_split(lo, hi, w_ref):
        return (jnp.dot(lo, w_ref[0:DH, :], preferred_element_type=f32)
                + jnp.dot(hi, w_ref[DH:D, :], preferred_element_type=f32))

    s1 = matmul_split(a0l_ref[...] * den, a0h_ref[...] * den, wps_ref) * inv
    s1 = s1 * jax.nn.sigmoid(s1)
    s2 = jnp.dot(s1, wos_ref[...], preferred_element_type=f32) * inv
    sc = jnp.dot(nf_ref[...], wsc_ref[...], preferred_element_type=f32) * inv
    out_ref[:, 0:D] = sc + s2

    rows = lax.broadcasted_iota(jnp.int32, (D, 3 * D), 0)
    cols = lax.broadcasted_iota(jnp.int32, (D, 3 * D), 1)
    outv = jnp.zeros((out_ref.shape[0], 3 * D), f32)
    for i, (lo_ref, hi_ref) in enumerate(
            ((a1l_ref, a1h_ref), (a2l_ref, a2h_ref), (a3l_ref, a3h_ref))):
        v1 = matmul_split(lo_ref[...] * den, hi_ref[...] * den, wpv_ref) * inv
        v2 = jnp.dot(v1, wov_ref[...], preferred_element_type=f32) * inv
        perm = (cols == 3 * rows + i).astype(f32)
        outv = outv + jnp.dot(v2, perm, preferred_element_type=f32)
    out_ref[:, D:4 * D] = outv


def _tc_update(aggs, node_features, W_pre_s, W_pre_v, W_post_s, W_post_v, W_sc):
    bn = 1000
    grid = (N_NODES // bn,)
    half_spec = pl.BlockSpec((bn, DH), lambda i: (i, 0))
    row_spec = pl.BlockSpec((bn, D), lambda i: (i, 0))
    w_spec = pl.BlockSpec((D, D), lambda i: (0, 0))
    flat_aggs = [a for pair in aggs for a in pair]
    return pl.pallas_call(
        _tc_body,
        grid=grid,
        in_specs=[half_spec] * 8 + [row_spec] + [w_spec] * 5,
        out_specs=pl.BlockSpec((bn, 4 * D), lambda i: (i, 0)),
        out_shape=jax.ShapeDtypeStruct((N_NODES, 4 * D), jnp.float32),
    )(*flat_aggs, node_features, W_pre_s, W_pre_v, W_post_s, W_post_v, W_sc)


def kernel(node_features, positions, senders, receivers,
           W_pre_s, W_pre_v, W_post_s, W_post_v, W_sc):
    aggs = _sc_aggregate(node_features, positions, senders, receivers)
    return _tc_update(aggs, node_features,
                      W_pre_s, W_pre_v, W_post_s, W_post_v, W_sc)


# speedup vs baseline: 21.7040x; 1.0031x over previous
import jax
import jax.numpy as jnp
from jax import lax
from jax.experimental import pallas as pl
from jax.experimental.pallas import tpu as pltpu
from jax.experimental.pallas import tpu_sc as plsc

N_NODES = 10000
N_EDGES = 320000
D = 128
DH = 64

N_TILES = 16
EPT = 20480
E_PAD = EPT * N_TILES
BLK = 128
NBLK = EPT // BLK
GBLK = 16
GEDGE = GBLK * BLK
NGRP = NBLK // GBLK
N_PAD = N_NODES + 8
RPT = 632
RPT_LAST = N_NODES - 15 * RPT

_SQRT3 = 3.0 ** 0.5


def _rsqrt(x):
    i = lax.bitcast_convert_type(x, jnp.int32)
    i = jnp.int32(0x5F3759DF) - (i >> 1)
    y = lax.bitcast_convert_type(i, jnp.float32)
    for _ in range(3):
        y = y * (1.5 - 0.5 * x * y * y)
    return y


def _sc_body(nf0_hbm, nf1_hbm, px_hbm, py_hbm, pz_hbm, snd_hbm, rcv2_hbm,
             o00, o10, o01, o11, o20, o30, o21, o31,
             px_v, py_v, pz_v, sidxA, sidxB, ridxA, ridxB, wbuf,
             rows0, rows1, wrow0, wrow1, zbuf, acc,
             isem0, isem1, gsem0, gsem1, ssem0, ssem1):
    cid = lax.axis_index("c")
    sid = lax.axis_index("s")
    sweeps = ((0, o00, o10), (1, o01, o11), (0, o20, o30), (1, o21, o31))
    rows = (rows0, rows1)
    wrow = (wrow0, wrow1)
    sidx = (sidxA, sidxB)
    ridx = (ridxA, ridxB)
    isem = (isem0, isem1)
    gsem = (gsem0, gsem1)
    ssem = (ssem0, ssem1)

    pltpu.sync_copy(px_hbm, px_v)
    pltpu.sync_copy(py_hbm, py_v)
    pltpu.sync_copy(pz_hbm, pz_v)

    def _zero_row(i, _):
        for j in range(DH // 16):
            zbuf[i, pl.ds(j * 16, 16)] = jnp.zeros((16,), jnp.float32)
        return 0
    lax.fori_loop(0, 64, _zero_row, 0)

    zeros16i = jnp.zeros((16,), jnp.int32)
    ones16 = jnp.ones((16,), jnp.float32)
    r0 = pl.multiple_of(sid * RPT, 8)

    def _fire_idx(g, q):
        e0 = sid * EPT + g * GEDGE
        pltpu.async_copy(snd_hbm.at[pl.ds(e0, GEDGE)], sidx[q], isem[q])
        pltpu.async_copy(rcv2_hbm.at[pl.ds(sid * NBLK + g * GBLK, GBLK)],
                         ridx[q], isem[q])

    def _drain_idx(q):
        pltpu.make_async_copy(
            snd_hbm.at[pl.ds(0, GEDGE)], sidx[q], isem[q]).wait()
        pltpu.make_async_copy(
            rcv2_hbm.at[pl.ds(0, GBLK)], ridx[q], isem[q]).wait()

    def _fire_gather(nf_hbm, q, b_in, p):
        return pltpu.async_copy(
            nf_hbm.at[sidx[q].at[pl.ds(b_in * BLK, BLK)]], rows[p], gsem[p])

    def _drain_gather(nf_hbm, p):
        pltpu.make_async_copy(nf_hbm.at[pl.ds(0, BLK)], rows[p], gsem[p]).wait()

    def _drain_scatter(nf_hbm, p):
        pltpu.make_async_copy(nf_hbm.at[pl.ds(0, BLK)], wrow[p], ssem[p]).wait()

    for swp, (half, out_a, out_b) in enumerate(sweeps):
        nf_hbm = nf0_hbm if half == 0 else nf1_hbm

        for j in range(8):
            pltpu.sync_copy(zbuf, acc.at[pl.ds(r0 + j * 64, 64)])

        @pl.when(sid < N_TILES - 1)
        def _():
            pltpu.sync_copy(zbuf, acc.at[pl.ds(r0 + 512, 64)])
            pltpu.sync_copy(zbuf.at[pl.ds(0, RPT - 576)],
                            acc.at[pl.ds(r0 + 576, RPT - 576)])

        @pl.when(sid == N_TILES - 1)
        def _():
            pltpu.sync_copy(zbuf.at[pl.ds(0, N_PAD - 15 * RPT - 512)],
                            acc.at[pl.ds(15 * RPT + 512, N_PAD - 15 * RPT - 512)])

        _fire_idx(0, 0)
        plsc.subcore_barrier()

        def _group_pair(g2, _):
            for q in range(2):
                g = 2 * g2 + q
                if q == 0:
                    @pl.when(g2 > 0)
                    def _():
                        _drain_scatter(nf_hbm, 0)
                        _drain_scatter(nf_hbm, 1)
                else:
                    _drain_scatter(nf_hbm, 0)
                    _drain_scatter(nf_hbm, 1)
                _drain_idx(q)
                if q == 0:
                    _fire_idx(g + 1, 1)
                else:
                    @pl.when(g2 < NGRP // 2 - 1)
                    def _():
                        _fire_idx(g + 1, 0)
                _fire_gather(nf_hbm, q, 0, 0)
                _fire_gather(nf_hbm, q, 1, 1)

                def _pair(it, _, q=q):
                    for p in range(2):
                        b_in = 2 * it + p

                        def _weights(sel):
                            @plsc.parallel_loop(0, BLK // 16, 1, unroll=4)
                            def _w(i, q=q, b_in=b_in, p=p):
                                s16 = sidx[q][pl.ds(b_in * BLK + i * 16, 16)]
                                r16 = ridx[q][b_in, pl.ds(i * 16, 16)]
                                sx = plsc.load_gather(px_v, [s16])
                                sy = plsc.load_gather(py_v, [s16])
                                sz = plsc.load_gather(pz_v, [s16])
                                rx = plsc.load_gather(px_v, [r16])
                                ry = plsc.load_gather(py_v, [r16])
                                rz = plsc.load_gather(pz_v, [r16])
                                vx, vy, vz = rx - sx, ry - sy, rz - sz
                                rinv = _rsqrt(
                                    vx * vx + vy * vy + vz * vz + 1e-12) * _SQRT3
                                wbuf[p, pl.ds(i * 16, 16)] = sel(vx, vy, vz, rinv)

                        if swp < 2:
                            @pl.when(cid == 0)
                            def _(p=p):
                                @plsc.parallel_loop(0, BLK // 16, 1, unroll=4)
                                def _w1(i, p=p):
                                    wbuf[p, pl.ds(i * 16, 16)] = ones16

                            @pl.when(cid == 1)
                            def _():
                                _weights(lambda vx, vy, vz, rinv: vx * rinv)
                        else:
                            _weights(lambda vx, vy, vz, rinv: jnp.where(
                                cid == 0, vy * rinv, vz * rinv))

                        @pl.when(b_in >= 2)
                        def _(p=p):
                            _drain_scatter(nf_hbm, p)
                        _drain_gather(nf_hbm, p)

                        @plsc.parallel_loop(0, BLK, 1, unroll=8)
                        def _mul(e, p=p):
                            wsp = plsc.load_gather(
                                wbuf, [zeros16i + p, zeros16i + e])
                            for j in range(DH // 16):
                                wrow[p][e, pl.ds(j * 16, 16)] = (
                                    wsp * rows[p][e, pl.ds(j * 16, 16)])

                        pltpu.async_copy(wrow[p], acc.at[ridx[q].at[b_in]],
                                         ssem[p], add=True)

                        @pl.when(b_in + 2 < GBLK)
                        def _(q=q, p=p, b_in=b_in):
                            _fire_gather(nf_hbm, q, b_in + 2, p)
                    return 0

                lax.fori_loop(0, GBLK // 2, _pair, 0)
            return 0

        lax.fori_loop(0, NGRP // 2, _group_pair, 0)
        _drain_scatter(nf_hbm, 0)
        _drain_scatter(nf_hbm, 1)

        plsc.subcore_barrier()

        for this_cid, out_x in ((0, out_a), (1, out_b)):
            @pl.when((cid == this_cid) & (sid < N_TILES - 1))
            def _(out_x=out_x):
                pltpu.sync_copy(acc.at[pl.ds(r0, RPT)],
                                out_x.at[pl.ds(r0, RPT)])

            @pl.when((cid == this_cid) & (sid == N_TILES - 1))
            def _(out_x=out_x):
                pltpu.sync_copy(acc.at[pl.ds(15 * RPT, RPT_LAST)],
                                out_x.at[pl.ds(15 * RPT, RPT_LAST)])


def _sc_aggregate(node_features, positions, senders, receivers):
    pad = E_PAD - N_EDGES
    snd_p = jnp.concatenate([senders, jnp.zeros((pad,), jnp.int32)])
    rcv_p = jnp.concatenate([receivers, jnp.full((pad,), N_NODES, jnp.int32)])
    rcv2 = rcv_p.reshape(E_PAD // BLK, BLK)

    nf0 = node_features[:, :DH]
    nf1 = node_features[:, DH:]
    px, py, pz = positions[:, 0], positions[:, 1], positions[:, 2]
    mesh = plsc.VectorSubcoreMesh(core_axis_name="c", subcore_axis_name="s")
    f32 = jnp.float32
    agg_shape = jax.ShapeDtypeStruct((N_NODES, DH), f32)
    fn = pl.kernel(
        _sc_body,
        mesh=mesh,
        compiler_params=pltpu.CompilerParams(
            needs_layout_passes=False, use_tc_tiling_on_sc=False),
        out_type=tuple(agg_shape for _ in range(8)),
        scratch_types=[
            pltpu.VMEM((N_NODES,), f32),
            pltpu.VMEM((N_NODES,), f32),
            pltpu.VMEM((N_NODES,), f32),
            pltpu.VMEM((GEDGE,), jnp.int32),
            pltpu.VMEM((GEDGE,), jnp.int32),
            pltpu.VMEM((GBLK, BLK), jnp.int32),
            pltpu.VMEM((GBLK, BLK), jnp.int32),
            pltpu.VMEM((2, BLK), f32),
            pltpu.VMEM((BLK, DH), f32),
            pltpu.VMEM((BLK, DH), f32),
            pltpu.VMEM((BLK, DH), f32),
            pltpu.VMEM((BLK, DH), f32),
            pltpu.VMEM((64, DH), f32),
            pltpu.VMEM_SHARED((N_PAD, DH), f32),
            pltpu.SemaphoreType.DMA,
            pltpu.SemaphoreType.DMA,
            pltpu.SemaphoreType.DMA,
            pltpu.SemaphoreType.DMA,
            pltpu.SemaphoreType.DMA,
            pltpu.SemaphoreType.DMA,
        ],
    )
    o00, o10, o01, o11, o20, o30, o21, o31 = fn(
        nf0, nf1, px, py, pz, snd_p, rcv2)
    return ((o00, o01), (o10, o11), (o20, o21), (o30, o31))


def _tc_body(a0l_ref, a0h_ref, a1l_ref, a1h_ref, a2l_ref, a2h_ref,
             a3l_ref, a3h_ref, nf_ref,
             wps_ref, wpv_ref, wos_ref, wov_ref, wsc_ref, out_ref):
    inv = 1.0 / (D ** 0.5)
    den = 1.0 / 32.0
    f32 = jnp.float32

    def matmul_split(lo, hi, w_ref):
        return (jnp.dot(lo, w_ref[0:DH, :], preferred_element_type=f32)
                + jnp.dot(hi, w_ref[DH:D, :], preferred_element_type=f32))

    s1 = matmul_split(a0l_ref[...] * den, a0h_ref[...] * den, wps_ref) * inv
    s1 = s1 * jax.nn.sigmoid(s1)
    s2 = jnp.dot(s1, wos_ref[...], preferred_element_type=f32) * inv
    sc = jnp.dot(nf_ref[...], wsc_ref[...], preferred_element_type=f32) * inv
    out_ref[:, 0:D] = sc + s2

    rows = lax.broadcasted_iota(jnp.int32, (D, 3 * D), 0)
    cols = lax.broadcasted_iota(jnp.int32, (D, 3 * D), 1)
    outv = jnp.zeros((out_ref.shape[0], 3 * D), f32)
    for i, (lo_ref, hi_ref) in enumerate(
            ((a1l_ref, a1h_ref), (a2l_ref, a2h_ref), (a3l_ref, a3h_ref))):
        v1 = matmul_split(lo_ref[...] * den, hi_ref[...] * den, wpv_ref) * inv
        v2 = jnp.dot(v1, wov_ref[...], preferred_element_type=f32) * inv
        perm = (cols == 3 * rows + i).astype(f32)
        outv = outv + jnp.dot(v2, perm, preferred_element_type=f32)
    out_ref[:, D:4 * D] = outv


def _tc_update(aggs, node_features, W_pre_s, W_pre_v, W_post_s, W_post_v, W_sc):
    bn = 1000
    grid = (N_NODES // bn,)
    half_spec = pl.BlockSpec((bn, DH), lambda i: (i, 0))
    row_spec = pl.BlockSpec((bn, D), lambda i: (i, 0))
    w_spec = pl.BlockSpec((D, D), lambda i: (0, 0))
    flat_aggs = [a for pair in aggs for a in pair]
    return pl.pallas_call(
        _tc_body,
        grid=grid,
        in_specs=[half_spec] * 8 + [row_spec] + [w_spec] * 5,
        out_specs=pl.BlockSpec((bn, 4 * D), lambda i: (i, 0)),
        out_shape=jax.ShapeDtypeStruct((N_NODES, 4 * D), jnp.float32),
    )(*flat_aggs, node_features, W_pre_s, W_pre_v, W_post_s, W_post_v, W_sc)


def kernel(node_features, positions, senders, receivers,
           W_pre_s, W_pre_v, W_post_s, W_post_v, W_sc):
    aggs = _sc_aggregate(node_features, positions, senders, receivers)
    return _tc_update(aggs, node_features,
                      W_pre_s, W_pre_v, W_post_s, W_post_v, W_sc)


# speedup vs baseline: 38.7601x; 1.7858x over previous
import jax
import jax.numpy as jnp
from jax import lax
from jax.experimental import pallas as pl
from jax.experimental.pallas import tpu as pltpu
from jax.experimental.pallas import tpu_sc as plsc

N_NODES = 10000
N_EDGES = 320000
D = 128
DQ = 32

N_TILES = 16
EPT = 20480
E_PAD = EPT * N_TILES
BLK = 128
NBLK = EPT // BLK
GBLK = 16
GEDGE = GBLK * BLK
NGRP = NBLK // GBLK
N_PAD = N_NODES + 8
RPT = 632
RPT_LAST = N_NODES - 15 * RPT

_SQRT3 = 3.0 ** 0.5


def _rsqrt(x):
    i = lax.bitcast_convert_type(x, jnp.int32)
    i = jnp.int32(0x5F3759DF) - (i >> 1)
    y = lax.bitcast_convert_type(i, jnp.float32)
    for _ in range(3):
        y = y * (1.5 - 0.5 * x * y * y)
    return y


def _sc_body(nfq0, nfq1, nfq2, nfq3, px_hbm, py_hbm, pz_hbm, snd_hbm, rcv2_hbm,
             *rest):
    outs = rest[:16]
    (px_v, py_v, pz_v, sidxA, sidxB, ridxA, ridxB, wbufA, wbufB,
     rows0, rows1, wrowA0, wrowA1, wrowB0, wrowB1, zbuf, accA, accB,
     isem0, isem1, gsem0, gsem1, ssem0, ssem1) = rest[16:]
    cid = lax.axis_index("c")
    sid = lax.axis_index("s")
    nfq = (nfq0, nfq1, nfq2, nfq3)
    rows = (rows0, rows1)
    wrowA = (wrowA0, wrowA1)
    wrowB = (wrowB0, wrowB1)
    sidx = (sidxA, sidxB)
    ridx = (ridxA, ridxB)
    isem = (isem0, isem1)
    gsem = (gsem0, gsem1)
    ssem = (ssem0, ssem1)

    pltpu.sync_copy(px_hbm, px_v)
    pltpu.sync_copy(py_hbm, py_v)
    pltpu.sync_copy(pz_hbm, pz_v)

    def _zero_row(i, _):
        for j in range(DQ // 16):
            zbuf[i, pl.ds(j * 16, 16)] = jnp.zeros((16,), jnp.float32)
        return 0
    lax.fori_loop(0, 64, _zero_row, 0)

    zeros16i = jnp.zeros((16,), jnp.int32)
    r0 = pl.multiple_of(sid * RPT, 8)

    def _fire_idx(g, q):
        e0 = sid * EPT + g * GEDGE
        pltpu.async_copy(snd_hbm.at[pl.ds(e0, GEDGE)], sidx[q], isem[q])
        pltpu.async_copy(rcv2_hbm.at[pl.ds(sid * NBLK + g * GBLK, GBLK)],
                         ridx[q], isem[q])

    def _drain_idx(q):
        pltpu.make_async_copy(
            snd_hbm.at[pl.ds(0, GEDGE)], sidx[q], isem[q]).wait()
        pltpu.make_async_copy(
            rcv2_hbm.at[pl.ds(0, GBLK)], ridx[q], isem[q]).wait()

    def _fire_gather(t, q, b_in, p):
        @pl.when(cid == 0)
        def _():
            pltpu.async_copy(
                nfq[2 * (t % 2)].at[sidx[q].at[pl.ds(b_in * BLK, BLK)]],
                rows[p], gsem[p])

        @pl.when(cid == 1)
        def _():
            pltpu.async_copy(
                nfq[2 * (t % 2) + 1].at[sidx[q].at[pl.ds(b_in * BLK, BLK)]],
                rows[p], gsem[p])

    def _drain_gather(p):
        pltpu.make_async_copy(nfq0.at[pl.ds(0, BLK)], rows[p], gsem[p]).wait()

    def _drain_scatter(p):
        pltpu.make_async_copy(nfq0.at[pl.ds(0, BLK)], wrowA[p], ssem[p]).wait()
        pltpu.make_async_copy(nfq0.at[pl.ds(0, BLK)], wrowB[p], ssem[p]).wait()

    for t in range(4):
        klo = 0 if t < 2 else 2

        for acc in (accA, accB):
            for j in range(8):
                pltpu.sync_copy(zbuf, acc.at[pl.ds(r0 + j * 64, 64)])

            @pl.when(sid < N_TILES - 1)
            def _(acc=acc):
                pltpu.sync_copy(zbuf, acc.at[pl.ds(r0 + 512, 64)])
                pltpu.sync_copy(zbuf.at[pl.ds(0, RPT - 576)],
                                acc.at[pl.ds(r0 + 576, RPT - 576)])

            @pl.when(sid == N_TILES - 1)
            def _(acc=acc):
                pltpu.sync_copy(
                    zbuf.at[pl.ds(0, N_PAD - 15 * RPT - 512)],
                    acc.at[pl.ds(15 * RPT + 512, N_PAD - 15 * RPT - 512)])

        _fire_idx(0, 0)
        plsc.subcore_barrier()

        def _group_pair(g2, _):
            for q in range(2):
                g = 2 * g2 + q
                if q == 0:
                    @pl.when(g2 > 0)
                    def _():
                        _drain_scatter(0)
                        _drain_scatter(1)
                else:
                    _drain_scatter(0)
                    _drain_scatter(1)
                _drain_idx(q)
                if q == 0:
                    _fire_idx(g + 1, 1)
                else:
                    @pl.when(g2 < NGRP // 2 - 1)
                    def _():
                        _fire_idx(g + 1, 0)
                _fire_gather(t, q, 0, 0)
                _fire_gather(t, q, 1, 1)

                def _pair(it, _, q=q):
                    for p in range(2):
                        b_in = 2 * it + p

                        @plsc.parallel_loop(0, BLK // 16, 1, unroll=4)
                        def _w(i, q=q, b_in=b_in, p=p):
                            s16 = sidx[q][pl.ds(b_in * BLK + i * 16, 16)]
                            r16 = ridx[q][b_in, pl.ds(i * 16, 16)]
                            sx = plsc.load_gather(px_v, [s16])
                            sy = plsc.load_gather(py_v, [s16])
                            sz = plsc.load_gather(pz_v, [s16])
                            rx = plsc.load_gather(px_v, [r16])
                            ry = plsc.load_gather(py_v, [r16])
                            rz = plsc.load_gather(pz_v, [r16])
                            vx, vy, vz = rx - sx, ry - sy, rz - sz
                            rinv = _rsqrt(
                                vx * vx + vy * vy + vz * vz + 1e-12) * _SQRT3
                            if t < 2:
                                wbufB[p, pl.ds(i * 16, 16)] = vx * rinv
                            else:
                                wbufA[p, pl.ds(i * 16, 16)] = vy * rinv
                                wbufB[p, pl.ds(i * 16, 16)] = vz * rinv

                        @pl.when(b_in >= 2)
                        def _(p=p):
                            _drain_scatter(p)
                        _drain_gather(p)

                        @plsc.parallel_loop(0, BLK, 1, unroll=8)
                        def _mul(e, p=p):
                            if t < 2:
                                wspB = plsc.load_gather(
                                    wbufB, [zeros16i + p, zeros16i + e])
                                for j in range(DQ // 16):
                                    r = rows[p][e, pl.ds(j * 16, 16)]
                                    wrowA[p][e, pl.ds(j * 16, 16)] = r
                                    wrowB[p][e, pl.ds(j * 16, 16)] = wspB * r
                            else:
                                wspA = plsc.load_gather(
                                    wbufA, [zeros16i + p, zeros16i + e])
                                wspB = plsc.load_gather(
                                    wbufB, [zeros16i + p, zeros16i + e])
                                for j in range(DQ // 16):
                                    r = rows[p][e, pl.ds(j * 16, 16)]
                                    wrowA[p][e, pl.ds(j * 16, 16)] = wspA * r
                                    wrowB[p][e, pl.ds(j * 16, 16)] = wspB * r

                        pltpu.async_copy(wrowA[p], accA.at[ridx[q].at[b_in]],
                                         ssem[p], add=True)
                        pltpu.async_copy(wrowB[p], accB.at[ridx[q].at[b_in]],
                                         ssem[p], add=True)

                        @pl.when(b_in + 2 < GBLK)
                        def _(q=q, p=p, b_in=b_in):
                            _fire_gather(t, q, b_in + 2, p)
                    return 0

                lax.fori_loop(0, GBLK // 2, _pair, 0)
            return 0

        lax.fori_loop(0, NGRP // 2, _group_pair, 0)
        _drain_scatter(0)
        _drain_scatter(1)

        plsc.subcore_barrier()

        for this_cid in (0, 1):
            qt = 2 * (t % 2) + this_cid
            for acc, k in ((accA, klo), (accB, klo + 1)):
                out_x = outs[4 * k + qt]

                @pl.when((cid == this_cid) & (sid < N_TILES - 1))
                def _(acc=acc, out_x=out_x):
                    pltpu.sync_copy(acc.at[pl.ds(r0, RPT)],
                                    out_x.at[pl.ds(r0, RPT)])

                @pl.when((cid == this_cid) & (sid == N_TILES - 1))
                def _(acc=acc, out_x=out_x):
                    pltpu.sync_copy(acc.at[pl.ds(15 * RPT, RPT_LAST)],
                                    out_x.at[pl.ds(15 * RPT, RPT_LAST)])


def _sc_aggregate(node_features, positions, senders, receivers):
    pad = E_PAD - N_EDGES
    snd_p = jnp.concatenate([senders, jnp.zeros((pad,), jnp.int32)])
    rcv_p = jnp.concatenate([receivers, jnp.full((pad,), N_NODES, jnp.int32)])
    rcv2 = rcv_p.reshape(E_PAD // BLK, BLK)

    nfq = [node_features[:, i * DQ:(i + 1) * DQ] for i in range(4)]
    px, py, pz = positions[:, 0], positions[:, 1], positions[:, 2]
    mesh = plsc.VectorSubcoreMesh(core_axis_name="c", subcore_axis_name="s")
    f32 = jnp.float32
    agg_shape = jax.ShapeDtypeStruct((N_NODES, DQ), f32)
    fn = pl.kernel(
        _sc_body,
        mesh=mesh,
        compiler_params=pltpu.CompilerParams(
            needs_layout_passes=False, use_tc_tiling_on_sc=False),
        out_type=tuple(agg_shape for _ in range(16)),
        scratch_types=[
            pltpu.VMEM((N_NODES,), f32),
            pltpu.VMEM((N_NODES,), f32),
            pltpu.VMEM((N_NODES,), f32),
            pltpu.VMEM((GEDGE,), jnp.int32),
            pltpu.VMEM((GEDGE,), jnp.int32),
            pltpu.VMEM((GBLK, BLK), jnp.int32),
            pltpu.VMEM((GBLK, BLK), jnp.int32),
            pltpu.VMEM((2, BLK), f32),
            pltpu.VMEM((2, BLK), f32),
            pltpu.VMEM((BLK, DQ), f32),
            pltpu.VMEM((BLK, DQ), f32),
            pltpu.VMEM((BLK, DQ), f32),
            pltpu.VMEM((BLK, DQ), f32),
            pltpu.VMEM((BLK, DQ), f32),
            pltpu.VMEM((BLK, DQ), f32),
            pltpu.VMEM((64, DQ), f32),
            pltpu.VMEM_SHARED((N_PAD, DQ), f32),
            pltpu.VMEM_SHARED((N_PAD, DQ), f32),
            pltpu.SemaphoreType.DMA,
            pltpu.SemaphoreType.DMA,
            pltpu.SemaphoreType.DMA,
            pltpu.SemaphoreType.DMA,
            pltpu.SemaphoreType.DMA,
            pltpu.SemaphoreType.DMA,
        ],
    )
    outs = fn(*nfq, px, py, pz, snd_p, rcv2)
    return tuple(tuple(outs[4 * k + q] for q in range(4)) for k in range(4))


def _tc_body(*refs):
    aggs = refs[:16]
    nf_ref, wps_ref, wpv_ref, wos_ref, wov_ref, wsc_ref, out_ref = refs[16:]
    inv = 1.0 / (D ** 0.5)
    den = 1.0 / 32.0
    f32 = jnp.float32

    def matmul_split(k, w_ref, scale):
        acc = jnp.zeros((out_ref.shape[0], D), f32)
        for q in range(4):
            acc = acc + jnp.dot(aggs[4 * k + q][...] * scale,
                                w_ref[q * DQ:(q + 1) * DQ, :],
                                preferred_element_type=f32)
        return acc

    s1 = matmul_split(0, wps_ref, den) * inv
    s1 = s1 * jax.nn.sigmoid(s1)
    s2 = jnp.dot(s1, wos_ref[...], preferred_element_type=f32) * inv
    sc = jnp.dot(nf_ref[...], wsc_ref[...], preferred_element_type=f32) * inv
    out_ref[:, 0:D] = sc + s2

    rows = lax.broadcasted_iota(jnp.int32, (D, 3 * D), 0)
    cols = lax.broadcasted_iota(jnp.int32, (D, 3 * D), 1)
    outv = jnp.zeros((out_ref.shape[0], 3 * D), f32)
    for i in range(3):
        v1 = matmul_split(1 + i, wpv_ref, den) * inv
        v2 = jnp.dot(v1, wov_ref[...], preferred_element_type=f32) * inv
        perm = (cols == 3 * rows + i).astype(f32)
        outv = outv + jnp.dot(v2, perm, preferred_element_type=f32)
    out_ref[:, D:4 * D] = outv


def _tc_update(aggs, node_features, W_pre_s, W_pre_v, W_post_s, W_post_v, W_sc):
    bn = 1000
    grid = (N_NODES // bn,)
    q_spec = pl.BlockSpec((bn, DQ), lambda i: (i, 0))
    row_spec = pl.BlockSpec((bn, D), lambda i: (i, 0))
    w_spec = pl.BlockSpec((D, D), lambda i: (0, 0))
    flat_aggs = [a for quad in aggs for a in quad]
    return pl.pallas_call(
        _tc_body,
        grid=grid,
        in_specs=[q_spec] * 16 + [row_spec] + [w_spec] * 5,
        out_specs=pl.BlockSpec((bn, 4 * D), lambda i: (i, 0)),
        out_shape=jax.ShapeDtypeStruct((N_NODES, 4 * D), jnp.float32),
    )(*flat_aggs, node_features, W_pre_s, W_pre_v, W_post_s, W_post_v, W_sc)


def kernel(node_features, positions, senders, receivers,
           W_pre_s, W_pre_v, W_post_s, W_post_v, W_sc):
    aggs = _sc_aggregate(node_features, positions, senders, receivers)
    return _tc_update(aggs, node_features,
                      W_pre_s, W_pre_v, W_post_s, W_post_v, W_sc)


# speedup vs baseline: 42.3304x; 1.0921x over previous
import jax
import jax.numpy as jnp
from jax import lax
from jax.experimental import pallas as pl
from jax.experimental.pallas import tpu as pltpu
from jax.experimental.pallas import tpu_sc as plsc

N_NODES = 10000
N_EDGES = 320000
D = 128
DQ = 16

N_TILES = 16
EPT = 20480
E_PAD = EPT * N_TILES
BLK = 128
NBLK = EPT // BLK
GBLK = 16
GEDGE = GBLK * BLK
NGRP = NBLK // GBLK
N_PAD = N_NODES + 8
RPT = 632
RPT_LAST = N_NODES - 15 * RPT

_SQRT3 = 3.0 ** 0.5


def _rsqrt(x):
    i = lax.bitcast_convert_type(x, jnp.int32)
    i = jnp.int32(0x5F3759DF) - (i >> 1)
    y = lax.bitcast_convert_type(i, jnp.float32)
    for _ in range(3):
        y = y * (1.5 - 0.5 * x * y * y)
    return y


def _sc_body(*args):
    nfq = args[:8]
    px_hbm, py_hbm, pz_hbm, snd_hbm, rcv2_hbm = args[8:13]
    outs = args[13:13 + 32]
    (px_v, py_v, pz_v, sidxA, sidxB, ridxA, ridxB, wbuf1, wbuf2, wbuf3,
     rows0, rows1, wA0, wA1, wB0, wB1, wC0, wC1, wD0, wD1, zbuf,
     accA, accB, accC, accD,
     isem0, isem1, gsem0, gsem1, ssem0, ssem1) = args[13 + 32:]
    cid = lax.axis_index("c")
    sid = lax.axis_index("s")
    rows = (rows0, rows1)
    wrow = ((wA0, wA1), (wB0, wB1), (wC0, wC1), (wD0, wD1))
    accs = (accA, accB, accC, accD)
    sidx = (sidxA, sidxB)
    ridx = (ridxA, ridxB)
    isem = (isem0, isem1)
    gsem = (gsem0, gsem1)
    ssem = (ssem0, ssem1)

    pltpu.sync_copy(px_hbm, px_v)
    pltpu.sync_copy(py_hbm, py_v)
    pltpu.sync_copy(pz_hbm, pz_v)

    def _zero_row(i, _):
        zbuf[i, pl.ds(0, 16)] = jnp.zeros((16,), jnp.float32)
        return 0
    lax.fori_loop(0, 64, _zero_row, 0)

    zeros16i = jnp.zeros((16,), jnp.int32)
    r0 = pl.multiple_of(sid * RPT, 8)

    def _fire_idx(g, q):
        e0 = sid * EPT + g * GEDGE
        pltpu.async_copy(snd_hbm.at[pl.ds(e0, GEDGE)], sidx[q], isem[q])
        pltpu.async_copy(rcv2_hbm.at[pl.ds(sid * NBLK + g * GBLK, GBLK)],
                         ridx[q], isem[q])

    def _drain_idx(q):
        pltpu.make_async_copy(
            snd_hbm.at[pl.ds(0, GEDGE)], sidx[q], isem[q]).wait()
        pltpu.make_async_copy(
            rcv2_hbm.at[pl.ds(0, GBLK)], ridx[q], isem[q]).wait()

    def _fire_gather(t, q, b_in, p):
        @pl.when(cid == 0)
        def _():
            pltpu.async_copy(
                nfq[2 * t].at[sidx[q].at[pl.ds(b_in * BLK, BLK)]],
                rows[p], gsem[p])

        @pl.when(cid == 1)
        def _():
            pltpu.async_copy(
                nfq[2 * t + 1].at[sidx[q].at[pl.ds(b_in * BLK, BLK)]],
                rows[p], gsem[p])

    def _drain_gather(p):
        pltpu.make_async_copy(
            nfq[0].at[pl.ds(0, BLK)], rows[p], gsem[p]).wait()

    def _drain_scatter(p):
        for k in range(4):
            pltpu.make_async_copy(
                nfq[0].at[pl.ds(0, BLK)], wrow[k][p], ssem[p]).wait()

    for t in range(4):
        for acc in accs:
            for j in range(8):
                pltpu.sync_copy(zbuf, acc.at[pl.ds(r0 + j * 64, 64)])

            @pl.when(sid < N_TILES - 1)
            def _(acc=acc):
                pltpu.sync_copy(zbuf, acc.at[pl.ds(r0 + 512, 64)])
                pltpu.sync_copy(zbuf.at[pl.ds(0, RPT - 576)],
                                acc.at[pl.ds(r0 + 576, RPT - 576)])

            @pl.when(sid == N_TILES - 1)
            def _(acc=acc):
                pltpu.sync_copy(
                    zbuf.at[pl.ds(0, N_PAD - 15 * RPT - 512)],
                    acc.at[pl.ds(15 * RPT + 512, N_PAD - 15 * RPT - 512)])

        _fire_idx(0, 0)
        plsc.subcore_barrier()

        def _group_pair(g2, _):
            for q in range(2):
                g = 2 * g2 + q
                if q == 0:
                    @pl.when(g2 > 0)
                    def _():
                        _drain_scatter(0)
                        _drain_scatter(1)
                else:
                    _drain_scatter(0)
                    _drain_scatter(1)
                _drain_idx(q)
                if q == 0:
                    _fire_idx(g + 1, 1)
                else:
                    @pl.when(g2 < NGRP // 2 - 1)
                    def _():
                        _fire_idx(g + 1, 0)
                _fire_gather(t, q, 0, 0)
                _fire_gather(t, q, 1, 1)

                def _pair(it, _, q=q):
                    for p in range(2):
                        b_in = 2 * it + p

                        @plsc.parallel_loop(0, BLK // 16, 1, unroll=4)
                        def _w(i, q=q, b_in=b_in, p=p):
                            s16 = sidx[q][pl.ds(b_in * BLK + i * 16, 16)]
                            r16 = ridx[q][b_in, pl.ds(i * 16, 16)]
                            sx = plsc.load_gather(px_v, [s16])
                            sy = plsc.load_gather(py_v, [s16])
                            sz = plsc.load_gather(pz_v, [s16])
                            rx = plsc.load_gather(px_v, [r16])
                            ry = plsc.load_gather(py_v, [r16])
                            rz = plsc.load_gather(pz_v, [r16])
                            vx, vy, vz = rx - sx, ry - sy, rz - sz
                            rinv = _rsqrt(
                                vx * vx + vy * vy + vz * vz + 1e-12) * _SQRT3
                            wbuf1[p, pl.ds(i * 16, 16)] = vx * rinv
                            wbuf2[p, pl.ds(i * 16, 16)] = vy * rinv
                            wbuf3[p, pl.ds(i * 16, 16)] = vz * rinv

                        @pl.when(b_in >= 2)
                        def _(p=p):
                            _drain_scatter(p)
                        _drain_gather(p)

                        @plsc.parallel_loop(0, BLK, 1, unroll=8)
                        def _mul(e, p=p):
                            w1 = plsc.load_gather(
                                wbuf1, [zeros16i + p, zeros16i + e])
                            w2 = plsc.load_gather(
                                wbuf2, [zeros16i + p, zeros16i + e])
                            w3 = plsc.load_gather(
                                wbuf3, [zeros16i + p, zeros16i + e])
                            r = rows[p][e, pl.ds(0, 16)]
                            wrow[0][p][e, pl.ds(0, 16)] = r
                            wrow[1][p][e, pl.ds(0, 16)] = w1 * r
                            wrow[2][p][e, pl.ds(0, 16)] = w2 * r
                            wrow[3][p][e, pl.ds(0, 16)] = w3 * r

                        for k in range(4):
                            pltpu.async_copy(
                                wrow[k][p], accs[k].at[ridx[q].at[b_in]],
                                ssem[p], add=True)

                        @pl.when(b_in + 2 < GBLK)
                        def _(q=q, p=p, b_in=b_in):
                            _fire_gather(t, q, b_in + 2, p)
                    return 0

                lax.fori_loop(0, GBLK // 2, _pair, 0)
            return 0

        lax.fori_loop(0, NGRP // 2, _group_pair, 0)
        _drain_scatter(0)
        _drain_scatter(1)

        plsc.subcore_barrier()

        for this_cid in (0, 1):
            e = 2 * t + this_cid
            for k in range(4):
                out_x = outs[8 * k + e]

                @pl.when((cid == this_cid) & (sid < N_TILES - 1))
                def _(k=k, out_x=out_x):
                    pltpu.sync_copy(accs[k].at[pl.ds(r0, RPT)],
                                    out_x.at[pl.ds(r0, RPT)])

                @pl.when((cid == this_cid) & (sid == N_TILES - 1))
                def _(k=k, out_x=out_x):
                    pltpu.sync_copy(accs[k].at[pl.ds(15 * RPT, RPT_LAST)],
                                    out_x.at[pl.ds(15 * RPT, RPT_LAST)])


def _sc_aggregate(node_features, positions, senders, receivers):
    pad = E_PAD - N_EDGES
    snd_p = jnp.concatenate([senders, jnp.zeros((pad,), jnp.int32)])
    rcv_p = jnp.concatenate([receivers, jnp.full((pad,), N_NODES, jnp.int32)])
    rcv2 = rcv_p.reshape(E_PAD // BLK, BLK)

    nfq = [node_features[:, i * DQ:(i + 1) * DQ] for i in range(8)]
    px, py, pz = positions[:, 0], positions[:, 1], positions[:, 2]
    mesh = plsc.VectorSubcoreMesh(core_axis_name="c", subcore_axis_name="s")
    f32 = jnp.float32
    agg_shape = jax.ShapeDtypeStruct((N_NODES, DQ), f32)
    fn = pl.kernel(
        _sc_body,
        mesh=mesh,
        compiler_params=pltpu.CompilerParams(
            needs_layout_passes=False, use_tc_tiling_on_sc=False),
        out_type=tuple(agg_shape for _ in range(32)),
        scratch_types=[
            pltpu.VMEM((N_NODES,), f32),
            pltpu.VMEM((N_NODES,), f32),
            pltpu.VMEM((N_NODES,), f32),
            pltpu.VMEM((GEDGE,), jnp.int32),
            pltpu.VMEM((GEDGE,), jnp.int32),
            pltpu.VMEM((GBLK, BLK), jnp.int32),
            pltpu.VMEM((GBLK, BLK), jnp.int32),
            pltpu.VMEM((2, BLK), f32),
            pltpu.VMEM((2, BLK), f32),
            pltpu.VMEM((2, BLK), f32),
            pltpu.VMEM((BLK, DQ), f32),
            pltpu.VMEM((BLK, DQ), f32),
            pltpu.VMEM((BLK, DQ), f32),
            pltpu.VMEM((BLK, DQ), f32),
            pltpu.VMEM((BLK, DQ), f32),
            pltpu.VMEM((BLK, DQ), f32),
            pltpu.VMEM((BLK, DQ), f32),
            pltpu.VMEM((BLK, DQ), f32),
            pltpu.VMEM((BLK, DQ), f32),
            pltpu.VMEM((BLK, DQ), f32),
            pltpu.VMEM((64, DQ), f32),
            pltpu.VMEM_SHARED((N_PAD, DQ), f32),
            pltpu.VMEM_SHARED((N_PAD, DQ), f32),
            pltpu.VMEM_SHARED((N_PAD, DQ), f32),
            pltpu.VMEM_SHARED((N_PAD, DQ), f32),
            pltpu.SemaphoreType.DMA,
            pltpu.SemaphoreType.DMA,
            pltpu.SemaphoreType.DMA,
            pltpu.SemaphoreType.DMA,
            pltpu.SemaphoreType.DMA,
            pltpu.SemaphoreType.DMA,
        ],
    )
    outs = fn(*nfq, px, py, pz, snd_p, rcv2)
    return tuple(tuple(outs[8 * k + e] for e in range(8)) for k in range(4))


def _tc_body(*refs):
    aggs = refs[:32]
    nf_ref, wps_ref, wpv_ref, wos_ref, wov_ref, wsc_ref, out_ref = refs[32:]
    inv = 1.0 / (D ** 0.5)
    den = 1.0 / 32.0
    f32 = jnp.float32

    def matmul_split(k, w_ref, scale):
        acc = jnp.zeros((out_ref.shape[0], D), f32)
        for e in range(8):
            acc = acc + jnp.dot(aggs[8 * k + e][...] * scale,
                                w_ref[e * DQ:(e + 1) * DQ, :],
                                preferred_element_type=f32)
        return acc

    s1 = matmul_split(0, wps_ref, den) * inv
    s1 = s1 * jax.nn.sigmoid(s1)
    s2 = jnp.dot(s1, wos_ref[...], preferred_element_type=f32) * inv
    sc = jnp.dot(nf_ref[...], wsc_ref[...], preferred_element_type=f32) * inv
    out_ref[:, 0:D] = sc + s2

    rows = lax.broadcasted_iota(jnp.int32, (D, 3 * D), 0)
    cols = lax.broadcasted_iota(jnp.int32, (D, 3 * D), 1)
    outv = jnp.zeros((out_ref.shape[0], 3 * D), f32)
    for i in range(3):
        v1 = matmul_split(1 + i, wpv_ref, den) * inv
        v2 = jnp.dot(v1, wov_ref[...], preferred_element_type=f32) * inv
        perm = (cols == 3 * rows + i).astype(f32)
        outv = outv + jnp.dot(v2, perm, preferred_element_type=f32)
    out_ref[:, D:4 * D] = outv


def _tc_update(aggs, node_features, W_pre_s, W_pre_v, W_post_s, W_post_v, W_sc):
    bn = 1000
    grid = (N_NODES // bn,)
    q_spec = pl.BlockSpec((bn, DQ), lambda i: (i, 0))
    row_spec = pl.BlockSpec((bn, D), lambda i: (i, 0))
    w_spec = pl.BlockSpec((D, D), lambda i: (0, 0))
    flat_aggs = [a for tup in aggs for a in tup]
    return pl.pallas_call(
        _tc_body,
        grid=grid,
        in_specs=[q_spec] * 32 + [row_spec] + [w_spec] * 5,
        out_specs=pl.BlockSpec((bn, 4 * D), lambda i: (i, 0)),
        out_shape=jax.ShapeDtypeStruct((N_NODES, 4 * D), jnp.float32),
    )(*flat_aggs, node_features, W_pre_s, W_pre_v, W_post_s, W_post_v, W_sc)


def kernel(node_features, positions, senders, receivers,
           W_pre_s, W_pre_v, W_post_s, W_post_v, W_sc):
    aggs = _sc_aggregate(node_features, positions, senders, receivers)
    return _tc_update(aggs, node_features,
                      W_pre_s, W_pre_v, W_post_s, W_post_v, W_sc)
